# baseline (speedup 1.0000x reference)
import jax
import jax.numpy as jnp
import numpy as np
from jax import lax
from jax.experimental import pallas as pl
from jax.experimental.pallas import tpu as pltpu

B, N = 1024, 200
BN = B * N
TB = 32
T = TB * N
OFFS = (0, 26, 44, 69, 77)


def _tcol_np(ci):
    return ((ci >= OFFS[1]).astype(np.int32) + (ci >= OFFS[2])
            + (ci >= OFFS[3]) + (ci >= OFFS[4]))


def _fold_body(stWb_ref, W096_ref, A10_ref, Wemb_ref, Tcat_ref, b0_ref,
               band_ref, M_ref, Temb_ref, bias_ref):
    A3 = jnp.dot(stWb_ref[...], W096_ref[...],
                 preferred_element_type=jnp.float32)
    Temb_ref[...] = jnp.dot(Tcat_ref[...] * band_ref[...], Wemb_ref[...],
                            preferred_element_type=jnp.float32)
    M_ref[...] = jnp.concatenate(
        [A3[0:3, :], A10_ref[...], jnp.zeros((5, 128), jnp.float32)], axis=0)
    bias_ref[...] = A3[3:4, :] + b0_ref[...]


def _body(fT_ref, M_ref, Temb_ref, bias_ref, P_ref, cmp_ref, out_ref):
    x = fT_ref[...]
    dn = (((0,), (0,)), ((), ()))
    u = lax.dot_general(x, P_ref[...], dn,
                        preferred_element_type=jnp.float32)
    mh = (u == cmp_ref[...]).astype(jnp.float32)
    y = (lax.dot_general(x, M_ref[...], dn,
                         preferred_element_type=jnp.float32)
         + jnp.dot(mh, Temb_ref[...], preferred_element_type=jnp.float32)
         + bias_ref[...])
    out_ref[...] = jnp.maximum(y, 0.0).reshape(TB, N, 128)


@jax.jit
def kernel(st, feat, st_W, st_b, wind_tab, wth_tab, hour_tab, wk_tab,
           hol_tab, W0, b0):
    fT = jnp.concatenate([st.reshape(BN, 3), feat.reshape(BN, 15)],
                         axis=1).T
    stWb = jnp.concatenate([st_W, st_b[None, :],
                            jnp.zeros((4, 96), jnp.float32)], axis=0)
    W096 = W0[0:96, :]
    A10 = W0[96:106, :]
    Wemb = W0[106:186, :]
    Tcat = jnp.concatenate([wind_tab, wth_tab, hour_tab, wk_tab, hol_tab],
                           axis=0)
    Tcat_rep = jnp.tile(Tcat, (1, 5))
    b0r = b0[None, :]

    ci = np.arange(80)[None, :]
    t = _tcol_np(ci)
    tr = _tcol_np(np.arange(80)[:, None])
    band = ((ci >= 16 * tr) & (ci < 16 * tr + 16)).astype(np.float32)
    P = (np.broadcast_to(np.arange(18)[:, None], (18, 80))
         == (t + 13)).astype(np.float32)
    cmp = (ci - np.array(OFFS)[t]).astype(np.float32)

    M, Temb, bias = pl.pallas_call(
        _fold_body,
        in_specs=[pl.BlockSpec(s, lambda: (0, 0))
                  for s in [(8, 96), (96, 128), (10, 128), (80, 128),
                            (80, 80), (1, 128), (80, 80)]],
        out_specs=[pl.BlockSpec(s, lambda: (0, 0))
                   for s in [(18, 128), (80, 128), (1, 128)]],
        out_shape=[jax.ShapeDtypeStruct((18, 128), jnp.float32),
                   jax.ShapeDtypeStruct((80, 128), jnp.float32),
                   jax.ShapeDtypeStruct((1, 128), jnp.float32)],
    )(stWb, W096, A10, Wemb, Tcat_rep, b0r, jnp.asarray(band))

    out = pl.pallas_call(
        _body,
        grid=(B // TB,),
        in_specs=[
            pl.BlockSpec((18, T), lambda i: (0, i)),
            pl.BlockSpec((18, 128), lambda i: (0, 0)),
            pl.BlockSpec((80, 128), lambda i: (0, 0)),
            pl.BlockSpec((1, 128), lambda i: (0, 0)),
            pl.BlockSpec((18, 80), lambda i: (0, 0)),
            pl.BlockSpec((1, 80), lambda i: (0, 0)),
        ],
        out_specs=pl.BlockSpec((TB, N, 128), lambda i: (i, 0, 0)),
        out_shape=jax.ShapeDtypeStruct((B, N, 128), jnp.float32),
        compiler_params=pltpu.CompilerParams(
            dimension_semantics=("parallel",)),
    )(fT, M, Temb, bias, jnp.asarray(P), jnp.asarray(cmp))
    return out
